# baseline (speedup 1.0000x reference)
import jax
import jax.numpy as jnp
from jax.experimental import pallas as pl
from jax.experimental.pallas import tpu as pltpu

CAP = 65536
X_DIM = 128
Y0, Y1 = 32, 32
Y_FLAT = Y0 * Y1
ITER_R = CAP // 128


def _scatter_body(pos_ref, cnt_ref, srow_any, crow_any, it_in,
                  sb_in, cb_in, sb_out, cb_out, it_out, sem_rows):
    pos = pos_ref[0]
    cnt = cnt_ref[0]

    row_s = pltpu.make_async_copy(srow_any, sb_out.at[pl.ds(pos, 1)],
                                  sem_rows.at[0])
    row_c = pltpu.make_async_copy(crow_any, cb_out.at[pl.ds(pos, 1)],
                                  sem_rows.at[1])
    row_s.start()
    row_c.start()

    r = pos // 128
    c = pos - r * 128
    row_ids = jax.lax.broadcasted_iota(jnp.int32, (ITER_R, 128), 0)
    col_ids = jax.lax.broadcasted_iota(jnp.int32, (ITER_R, 128), 1)
    hit = (row_ids == r) & (col_ids == c)
    it_out[...] = jnp.where(hit, cnt, it_in[...])

    row_s.wait()
    row_c.wait()


def kernel(state_buffer, cam_data_buffer, iter_buffer, position, state,
           cam_data, count):
    pos2 = position.reshape(1)
    cnt2 = count.reshape(1)
    srow = state.reshape(1, X_DIM)
    crow = cam_data.reshape(1, Y0, Y1)
    iter2d = iter_buffer.reshape(ITER_R, 128)

    out_sb, out_cb, out_it = pl.pallas_call(
        _scatter_body,
        in_specs=[
            pl.BlockSpec(memory_space=pltpu.SMEM),
            pl.BlockSpec(memory_space=pltpu.SMEM),
            pl.BlockSpec(memory_space=pl.ANY),
            pl.BlockSpec(memory_space=pl.ANY),
            pl.BlockSpec(memory_space=pltpu.VMEM),
            pl.BlockSpec(memory_space=pl.ANY),
            pl.BlockSpec(memory_space=pl.ANY),
        ],
        out_specs=[
            pl.BlockSpec(memory_space=pl.ANY),
            pl.BlockSpec(memory_space=pl.ANY),
            pl.BlockSpec(memory_space=pltpu.VMEM),
        ],
        out_shape=[
            jax.ShapeDtypeStruct((CAP, X_DIM), jnp.float32),
            jax.ShapeDtypeStruct((CAP, Y0, Y1), jnp.float32),
            jax.ShapeDtypeStruct((ITER_R, 128), jnp.int32),
        ],
        scratch_shapes=[
            pltpu.SemaphoreType.DMA((2,)),
        ],
        input_output_aliases={5: 0, 6: 1},
    )(pos2, cnt2, srow, crow, iter2d, state_buffer, cam_data_buffer)

    new_position = jnp.remainder(position + 1, CAP)
    full_buffer = (position + 1) == CAP
    return (out_sb, out_cb, out_it.reshape(CAP),
            new_position, full_buffer)


# speedup vs baseline: 7.2986x; 7.2986x over previous
import jax
import jax.numpy as jnp
from jax.experimental import pallas as pl
from jax.experimental.pallas import tpu as pltpu

CAP = 65536
X_DIM = 128
Y0, Y1 = 32, 32

GRID = 64
CH = CAP // GRID


def _push_body(pos_ref, cnt_ref, srow_ref, crow_ref, sb_in, cb_in, it_in,
               sb_out, cb_out, it_out):
    i = pl.program_id(0)
    base = i * CH
    pos = pos_ref[0]
    cnt = cnt_ref[0]
    local = pos - base
    in_range = (pos >= base) & (pos < base + CH)

    sb_out[...] = sb_in[...]

    @pl.when(in_range)
    def _cam_sel():
        lane = jax.lax.broadcasted_iota(jnp.int32, (Y0, Y1, CH), 2)
        cb_out[...] = jnp.where(lane == local, crow_ref[...], cb_in[...])

    @pl.when(jnp.logical_not(in_range))
    def _cam_copy():
        cb_out[...] = cb_in[...]

    it_out[...] = it_in[...]

    @pl.when(in_range)
    def _overwrite():
        sb_out[pl.ds(local, 1), :] = srow_ref[...]
        col = jax.lax.broadcasted_iota(jnp.int32, (1, 1, CH), 2)
        it_out[...] = jnp.where(col == local, cnt, it_in[...])


def kernel(state_buffer, cam_data_buffer, iter_buffer, position, state,
           cam_data, count):
    pos2 = position.reshape(1)
    cnt2 = count.reshape(1)
    srow = state.reshape(1, X_DIM)
    crow = cam_data.reshape(Y0, Y1, 1)
    cam_t = jax.lax.transpose(cam_data_buffer, (1, 2, 0))
    iter3d = iter_buffer.reshape(GRID, 1, CH)

    out_sb, out_cb, out_it = pl.pallas_call(
        _push_body,
        grid=(GRID,),
        in_specs=[
            pl.BlockSpec(memory_space=pltpu.SMEM),
            pl.BlockSpec(memory_space=pltpu.SMEM),
            pl.BlockSpec((1, X_DIM), lambda i: (0, 0)),
            pl.BlockSpec((Y0, Y1, 1), lambda i: (0, 0, 0)),
            pl.BlockSpec((CH, X_DIM), lambda i: (i, 0)),
            pl.BlockSpec((Y0, Y1, CH), lambda i: (0, 0, i)),
            pl.BlockSpec((1, 1, CH), lambda i: (i, 0, 0)),
        ],
        out_specs=[
            pl.BlockSpec((CH, X_DIM), lambda i: (i, 0)),
            pl.BlockSpec((Y0, Y1, CH), lambda i: (0, 0, i)),
            pl.BlockSpec((1, 1, CH), lambda i: (i, 0, 0)),
        ],
        out_shape=[
            jax.ShapeDtypeStruct((CAP, X_DIM), jnp.float32),
            jax.ShapeDtypeStruct((Y0, Y1, CAP), jnp.float32),
            jax.ShapeDtypeStruct((GRID, 1, CH), jnp.int32),
        ],
        compiler_params=pltpu.CompilerParams(
            dimension_semantics=("arbitrary",),
        ),
    )(pos2, cnt2, srow, crow, state_buffer, cam_t, iter3d)

    new_position = jnp.remainder(position + 1, CAP)
    full_buffer = (position + 1) == CAP
    return (out_sb, jax.lax.transpose(out_cb, (2, 0, 1)),
            out_it.reshape(CAP), new_position, full_buffer)


# speedup vs baseline: 7.4464x; 1.0202x over previous
import jax
import jax.numpy as jnp
from jax.experimental import pallas as pl
from jax.experimental.pallas import tpu as pltpu

CAP = 65536
X_DIM = 128
Y0, Y1 = 32, 32

GRID = 32
CH = CAP // GRID


def _push_body(pos_ref, cnt_ref, srow_ref, crow_ref, sb_in, cb_in, it_in,
               sb_out, cb_out, it_out):
    i = pl.program_id(0)
    base = i * CH
    pos = pos_ref[0]
    cnt = cnt_ref[0]
    local = pos - base
    in_range = (pos >= base) & (pos < base + CH)

    sb_out[...] = sb_in[...]

    @pl.when(in_range)
    def _cam_sel():
        lane = jax.lax.broadcasted_iota(jnp.int32, (Y0, Y1, CH), 2)
        cb_out[...] = jnp.where(lane == local, crow_ref[...], cb_in[...])

    @pl.when(jnp.logical_not(in_range))
    def _cam_copy():
        cb_out[...] = cb_in[...]

    it_out[...] = it_in[...]

    @pl.when(in_range)
    def _overwrite():
        sb_out[pl.ds(local, 1), :] = srow_ref[...]
        col = jax.lax.broadcasted_iota(jnp.int32, (1, 1, CH), 2)
        it_out[...] = jnp.where(col == local, cnt, it_in[...])


def kernel(state_buffer, cam_data_buffer, iter_buffer, position, state,
           cam_data, count):
    pos2 = position.reshape(1)
    cnt2 = count.reshape(1)
    srow = state.reshape(1, X_DIM)
    crow = cam_data.reshape(Y0, Y1, 1)
    cam_t = jax.lax.transpose(cam_data_buffer, (1, 2, 0))
    iter3d = iter_buffer.reshape(GRID, 1, CH)

    out_sb, out_cb, out_it = pl.pallas_call(
        _push_body,
        grid=(GRID,),
        in_specs=[
            pl.BlockSpec(memory_space=pltpu.SMEM),
            pl.BlockSpec(memory_space=pltpu.SMEM),
            pl.BlockSpec((1, X_DIM), lambda i: (0, 0)),
            pl.BlockSpec((Y0, Y1, 1), lambda i: (0, 0, 0)),
            pl.BlockSpec((CH, X_DIM), lambda i: (i, 0)),
            pl.BlockSpec((Y0, Y1, CH), lambda i: (0, 0, i)),
            pl.BlockSpec((1, 1, CH), lambda i: (i, 0, 0)),
        ],
        out_specs=[
            pl.BlockSpec((CH, X_DIM), lambda i: (i, 0)),
            pl.BlockSpec((Y0, Y1, CH), lambda i: (0, 0, i)),
            pl.BlockSpec((1, 1, CH), lambda i: (i, 0, 0)),
        ],
        out_shape=[
            jax.ShapeDtypeStruct((CAP, X_DIM), jnp.float32),
            jax.ShapeDtypeStruct((Y0, Y1, CAP), jnp.float32),
            jax.ShapeDtypeStruct((GRID, 1, CH), jnp.int32),
        ],
        compiler_params=pltpu.CompilerParams(
            dimension_semantics=("arbitrary",),
        ),
    )(pos2, cnt2, srow, crow, state_buffer, cam_t, iter3d)

    new_position = jnp.remainder(position + 1, CAP)
    full_buffer = (position + 1) == CAP
    return (out_sb, jax.lax.transpose(out_cb, (2, 0, 1)),
            out_it.reshape(CAP), new_position, full_buffer)
